# baseline (speedup 1.0000x reference)
import functools

import jax
import jax.numpy as jnp
from jax import lax
from jax.experimental import pallas as pl
from jax.experimental.pallas import tpu as pltpu
from jax.experimental.pallas import tpu_sc as plsc

N = 1024
E = 32768
D_IN = 128
H = 64
ALPHA = 0.7
EPS = 1e-16
NEG_SLOPE = 0.2


def _gcn_dense(adj, dinv, h):
    return dinv * jnp.dot(adj, dinv * h, preferred_element_type=jnp.float32)


def _gat_prep(hg, a_s, a_d, we, ae, ew2d):
    als = jnp.dot(hg, a_s, preferred_element_type=jnp.float32)
    ald = jnp.dot(hg, a_d, preferred_element_type=jnp.float32)
    c = jnp.dot(we, ae, preferred_element_type=jnp.float32)
    c0 = c[0, 0]
    mew = jnp.mean(ew2d)
    maxew = jnp.max(ew2d)
    minew = jnp.min(ew2d)
    max_ale = jnp.maximum(jnp.maximum(c0 * maxew, c0 * minew), c0 * mew)
    cbound = jnp.maximum(jnp.max(als) + jnp.max(ald) + max_ale, 0.0)
    misc = jnp.concatenate(
        [
            jnp.full((1, 16), c0, jnp.float32),
            jnp.full((1, 16), c0 * mew, jnp.float32),
            jnp.full((1, 16), cbound, jnp.float32),
            jnp.zeros((5, 16), jnp.float32),
        ],
        axis=0,
    )
    return als, ald, misc


def _t1_body(x_ref, adj_ref, w1a_ref, b1a_ref, wg1_ref, as1_ref, ad1_ref,
             we1_ref, ae1_ref, ew_ref,
             xa1_ref, hg1_ref, als_ref, ald_ref, dinv_ref, misc_ref):
    adj = adj_ref[...]
    deg = jnp.sum(adj, axis=1, keepdims=True)
    dinv = jax.lax.rsqrt(deg)
    dinv_ref[...] = dinv
    x = x_ref[...]
    h1 = jnp.dot(x, w1a_ref[...], preferred_element_type=jnp.float32)
    xa1_ref[...] = jnp.maximum(_gcn_dense(adj, dinv, h1) + b1a_ref[...], 0.0)
    hg = jnp.dot(x, wg1_ref[...], preferred_element_type=jnp.float32)
    hg1_ref[...] = hg
    als, ald, misc = _gat_prep(hg, as1_ref[...], ad1_ref[...], we1_ref[...],
                               ae1_ref[...], ew_ref[...])
    als_ref[...] = als
    ald_ref[...] = ald
    misc_ref[...] = misc


def _t2_body(adj_ref, dinv_ref, xa1_ref, hg1_ref, e1_ref,
             w2a_ref, b2a_ref, wg2_ref, as2_ref, ad2_ref, we2_ref, ae2_ref,
             bg1_ref, ew_ref,
             xa2_ref, hg2_ref, als_ref, ald_ref, misc_ref):
    adj = adj_ref[...]
    dinv = dinv_ref[...]
    e1 = e1_ref[...]
    denom = jnp.sum(e1, axis=1, keepdims=True) + EPS
    agg = jnp.dot(e1, hg1_ref[...], preferred_element_type=jnp.float32)
    xd1 = jnp.maximum(agg / denom + bg1_ref[...], 0.0)
    h2 = jnp.dot(xa1_ref[...], w2a_ref[...], preferred_element_type=jnp.float32)
    xa2_ref[...] = jnp.maximum(_gcn_dense(adj, dinv, h2) + b2a_ref[...], 0.0)
    hg = jnp.dot(xd1, wg2_ref[...], preferred_element_type=jnp.float32)
    hg2_ref[...] = hg
    als, ald, misc = _gat_prep(hg, as2_ref[...], ad2_ref[...], we2_ref[...],
                               ae2_ref[...], ew_ref[...])
    als_ref[...] = als
    ald_ref[...] = ald
    misc_ref[...] = misc


def _t3_body(adj_ref, dinv_ref, xa2_ref, hg2_ref, e2_ref,
             bg2_ref, w3_ref, b3_ref, p1a_ref, p1b_ref, pb1_ref,
             a_ref, bt_ref):
    adj = adj_ref[...]
    dinv = dinv_ref[...]
    e2 = e2_ref[...]
    denom = jnp.sum(e2, axis=1, keepdims=True) + EPS
    agg = jnp.dot(e2, hg2_ref[...], preferred_element_type=jnp.float32)
    xd2 = jnp.maximum(agg / denom + bg2_ref[...], 0.0)
    xc = ALPHA * xa2_ref[...] + (1.0 - ALPHA) * xd2
    h3 = jnp.dot(xc, w3_ref[...], preferred_element_type=jnp.float32)
    hh = jnp.maximum(_gcn_dense(adj, dinv, h3) + b3_ref[...], 0.0)
    a_ref[...] = (
        jnp.dot(hh, p1a_ref[...], preferred_element_type=jnp.float32)
        + pb1_ref[...]
    )
    bt_ref[...] = jax.lax.dot_general(
        p1b_ref[...], hh, (((0,), (1,)), ((), ())),
        preferred_element_type=jnp.float32)


def _pair_body(a_ref, bt_ref, p2_ref, pb2_ref, out_ref):
    a = a_ref[...]
    bt = bt_ref[...]
    acc = jnp.zeros(out_ref.shape, jnp.float32)
    for k in range(H):
        acc = acc + jnp.maximum(a[:, k:k + 1] + bt[k:k + 1, :], 0.0) \
            * p2_ref[k:k + 1, :]
    z = acc + pb2_ref[...]
    out_ref[...] = 1.0 / (1.0 + jnp.exp(-z))


_NC = 2
_NS = 16
_NW = _NC * _NS
_EF = E + N
_EPT = _EF // _NS
_RPW = N // _NW
_CPB = 16


def _sc_mesh():
    return plsc.VectorSubcoreMesh(core_axis_name="c", subcore_axis_name="s")


def _scatter_phase(pv, valv, buf, wid, ones=None):
    row0 = wid * _RPW

    def body(k, _):
        for j in range(_CPB):
            off = k * (_CPB * 16) + j * 16
            p16 = pv[pl.ds(off, 16)]
            s16 = jnp.bitwise_and(p16, 0xFFFF)
            d16 = jnp.right_shift(p16, 16)
            rows = d16 - row0
            mask = (rows >= 0) & (rows < _RPW)
            v16 = ones if ones is not None else valv[pl.ds(off, 16)]
            plsc.addupdate_scatter(buf, [rows, s16], v16, mask=mask)
        return _

    lax.fori_loop(0, _EF // (_CPB * 16), body, 0, unroll=False)


def _adj_pallas(sd_packed, zeros2d):
    @functools.partial(
        pl.kernel,
        out_type=jax.ShapeDtypeStruct((N, N), jnp.float32),
        mesh=_sc_mesh(),
        compiler_params=pltpu.CompilerParams(needs_layout_passes=False),
        scratch_types=[
            pltpu.VMEM((_EF,), jnp.int32),
            pltpu.VMEM((_RPW, N), jnp.float32),
            pltpu.SemaphoreType.DMA,
        ],
    )
    def k(p_ref, z_ref, out_ref, pv, buf, sem):
        cid = lax.axis_index("c")
        sid = lax.axis_index("s")
        wid = sid * _NC + cid
        loads = [
            pltpu.async_copy(p_ref, pv, sem),
            pltpu.async_copy(z_ref.at[pl.ds(wid * _RPW, _RPW), :], buf, sem),
        ]
        for c in loads:
            c.wait()
        ones = jnp.full((16,), 1.0, jnp.float32)
        _scatter_phase(pv, None, buf, wid, ones=ones)
        pltpu.sync_copy(buf, out_ref.at[pl.ds(wid * _RPW, _RPW), :])

    return k(sd_packed, zeros2d)


def _gat_pallas(sd_packed, ew_pad, als, ald, misc, zeros2d):
    @functools.partial(
        pl.kernel,
        out_type=jax.ShapeDtypeStruct((N, N), jnp.float32),
        mesh=_sc_mesh(),
        compiler_params=pltpu.CompilerParams(needs_layout_passes=False),
        scratch_types=[
            pltpu.VMEM((_EF,), jnp.int32),
            pltpu.VMEM((_EF,), jnp.float32),
            pltpu.VMEM((_EPT,), jnp.float32),
            pltpu.VMEM((_EPT,), jnp.float32),
            pltpu.VMEM((_RPW, N), jnp.float32),
            pltpu.VMEM((N,), jnp.float32),
            pltpu.VMEM((N,), jnp.float32),
            pltpu.VMEM((128,), jnp.float32),
            pltpu.VMEM_SHARED((_EF,), jnp.float32),
            pltpu.SemaphoreType.DMA,
        ],
    )
    def k(p_ref, ew_ref, als_ref, ald_ref, misc_ref, z_ref, out_ref,
          pv, valv, myv, ewv, buf, alsv, aldv, miscv, xch, sem):
        cid = lax.axis_index("c")
        sid = lax.axis_index("s")
        wid = sid * _NC + cid
        ebase = sid * _EPT
        loads = [
            pltpu.async_copy(p_ref, pv, sem),
            pltpu.async_copy(ew_ref.at[pl.ds(ebase, _EPT)], ewv, sem),
            pltpu.async_copy(als_ref, alsv, sem),
            pltpu.async_copy(ald_ref, aldv, sem),
            pltpu.async_copy(misc_ref, miscv, sem),
            pltpu.async_copy(z_ref.at[pl.ds(wid * _RPW, _RPW), :], buf, sem),
        ]
        for c in loads:
            c.wait()
        c16 = miscv[pl.ds(0, 16)]
        selfe16 = miscv[pl.ds(16, 16)]
        cb16 = miscv[pl.ds(32, 16)]
        lanes = lax.iota(jnp.int32, 16)

        def val_body(k_, _):
            for j in range(12):
                off = k_ * (12 * 16) + j * 16
                p16 = pv[pl.ds(ebase + off, 16)]
                s16 = jnp.bitwise_and(p16, 0xFFFF)
                d16 = jnp.right_shift(p16, 16)
                ew16 = ewv[pl.ds(off, 16)]
                gi = ebase + off + lanes
                as16 = plsc.load_gather(alsv, [s16])
                ad16 = plsc.load_gather(aldv, [d16])
                ale = jnp.where(gi < E, c16 * ew16, selfe16)
                pre = as16 + ad16 + ale
                alpha = jnp.maximum(pre, NEG_SLOPE * pre)
                myv[pl.ds(off, 16)] = jnp.exp(alpha - cb16)
            return _

        lax.fori_loop(0, _EPT // (12 * 16), val_body, 0, unroll=False)
        pltpu.sync_copy(myv, xch.at[pl.ds(ebase, _EPT)])
        plsc.subcore_barrier()
        pltpu.sync_copy(xch, valv)

        _scatter_phase(pv, valv, buf, wid)
        pltpu.sync_copy(buf, out_ref.at[pl.ds(wid * _RPW, _RPW), :])

    return k(sd_packed, ew_pad, als, ald, misc, zeros2d)


def kernel(x, edge_index_above, edge_index_distance, edge_weights_distance,
           W1a, b1a, Wg1, as1, ad1, We1, ae1, bg1,
           W2a, b2a, Wg2, as2, ad2, We2, ae2, bg2,
           W3, b3, P1, pb1, P2, pb2):
    f32 = jnp.float32
    sa, da = edge_index_above[0], edge_index_above[1]
    sd, dd = edge_index_distance[0], edge_index_distance[1]
    loop = jnp.arange(N, dtype=sa.dtype)
    sa_f = jnp.concatenate([sa, loop])
    da_f = jnp.concatenate([da, loop])
    sd_f = jnp.concatenate([sd, loop])
    dd_f = jnp.concatenate([dd, loop])
    ew = edge_weights_distance
    ew2d = ew.reshape(E // 128, 128)
    ew_pad = jnp.concatenate([ew, jnp.zeros((N,), f32)])
    zeros2d = jnp.zeros((N, N), f32)
    sd_pack_a = jnp.left_shift(da_f, 16) | sa_f
    sd_pack_d = jnp.left_shift(dd_f, 16) | sd_f

    adj = _adj_pallas(sd_pack_a, zeros2d)

    b1a_2 = b1a.reshape(1, H)
    bg1_2 = bg1.reshape(1, H)
    b2a_2 = b2a.reshape(1, H)
    bg2_2 = bg2.reshape(1, H)
    b3_2 = b3.reshape(1, H)
    pb1_2 = pb1.reshape(1, H)
    pb2_2 = pb2.reshape(1, 1)
    as1_2, ad1_2 = as1.reshape(H, 1), ad1.reshape(H, 1)
    as2_2, ad2_2 = as2.reshape(H, 1), ad2.reshape(H, 1)
    ae1_2, ae2_2 = ae1.reshape(H, 1), ae2.reshape(H, 1)

    t1 = pl.pallas_call(
        _t1_body,
        out_shape=(
            jax.ShapeDtypeStruct((N, H), f32),
            jax.ShapeDtypeStruct((N, H), f32),
            jax.ShapeDtypeStruct((N, 1), f32),
            jax.ShapeDtypeStruct((N, 1), f32),
            jax.ShapeDtypeStruct((N, 1), f32),
            jax.ShapeDtypeStruct((8, 16), f32),
        ),
    )
    xa1, hg1, als1, ald1, dinv, misc1 = t1(
        x, adj, W1a, b1a_2, Wg1, as1_2, ad1_2, We1, ae1_2, ew2d)

    e1 = _gat_pallas(sd_pack_d, ew_pad, als1.reshape(N), ald1.reshape(N),
                     misc1.reshape(128), zeros2d)

    t2 = pl.pallas_call(
        _t2_body,
        out_shape=(
            jax.ShapeDtypeStruct((N, H), f32),
            jax.ShapeDtypeStruct((N, H), f32),
            jax.ShapeDtypeStruct((N, 1), f32),
            jax.ShapeDtypeStruct((N, 1), f32),
            jax.ShapeDtypeStruct((8, 16), f32),
        ),
    )
    xa2, hg2, als2, ald2, misc2 = t2(
        adj, dinv, xa1, hg1, e1, W2a, b2a_2, Wg2, as2_2, ad2_2, We2, ae2_2,
        bg1_2, ew2d)

    e2 = _gat_pallas(sd_pack_d, ew_pad, als2.reshape(N), ald2.reshape(N),
                     misc2.reshape(128), zeros2d)

    t3 = pl.pallas_call(
        _t3_body,
        out_shape=(
            jax.ShapeDtypeStruct((N, H), f32),
            jax.ShapeDtypeStruct((H, N), f32),
        ),
    )
    a_mat, bt_mat = t3(
        adj, dinv, xa2, hg2, e2, bg2_2, W3, b3_2, P1[:H], P1[H:], pb1_2)

    BI = 256
    pair = pl.pallas_call(
        _pair_body,
        grid=(N // BI,),
        in_specs=[
            pl.BlockSpec((BI, H), lambda i: (i, 0)),
            pl.BlockSpec((H, N), lambda i: (0, 0)),
            pl.BlockSpec((H, 1), lambda i: (0, 0)),
            pl.BlockSpec((1, 1), lambda i: (0, 0)),
        ],
        out_specs=pl.BlockSpec((BI, N), lambda i: (i, 0)),
        out_shape=jax.ShapeDtypeStruct((N, N), f32),
    )
    out2d = pair(a_mat, bt_mat, P2, pb2_2)
    return out2d.reshape(N * N)


# speedup vs baseline: 1.3475x; 1.3475x over previous
import functools

import jax
import jax.numpy as jnp
from jax import lax
from jax.experimental import pallas as pl
from jax.experimental.pallas import tpu as pltpu
from jax.experimental.pallas import tpu_sc as plsc

N = 1024
E = 32768
D_IN = 128
H = 64
ALPHA = 0.7
EPS = 1e-16
NEG_SLOPE = 0.2


def _gcn_dense(adj, dinv, h):
    return dinv * jnp.dot(adj, dinv * h, preferred_element_type=jnp.float32)


def _gat_prep(hg, a_s, a_d, we, ae, ew2d):
    als = jnp.dot(hg, a_s, preferred_element_type=jnp.float32)
    ald = jnp.dot(hg, a_d, preferred_element_type=jnp.float32)
    c = jnp.dot(we, ae, preferred_element_type=jnp.float32)
    c0 = c[0, 0]
    mew = jnp.mean(ew2d)
    maxew = jnp.max(ew2d)
    minew = jnp.min(ew2d)
    max_ale = jnp.maximum(jnp.maximum(c0 * maxew, c0 * minew), c0 * mew)
    cbound = jnp.maximum(jnp.max(als) + jnp.max(ald) + max_ale, 0.0)
    misc = jnp.concatenate(
        [
            jnp.full((1, 16), c0, jnp.float32),
            jnp.full((1, 16), c0 * mew, jnp.float32),
            jnp.full((1, 16), cbound, jnp.float32),
            jnp.zeros((5, 16), jnp.float32),
        ],
        axis=0,
    )
    return als, ald, misc


def _t1_body(x_ref, adj_ref, w1a_ref, b1a_ref, wg1_ref, as1_ref, ad1_ref,
             we1_ref, ae1_ref, ew_ref,
             xa1_ref, hg1_ref, als_ref, ald_ref, dinv_ref, misc_ref):
    adj = adj_ref[...]
    deg = jnp.sum(adj, axis=1, keepdims=True)
    dinv = jax.lax.rsqrt(deg)
    dinv_ref[...] = dinv
    x = x_ref[...]
    h1 = jnp.dot(x, w1a_ref[...], preferred_element_type=jnp.float32)
    xa1_ref[...] = jnp.maximum(_gcn_dense(adj, dinv, h1) + b1a_ref[...], 0.0)
    hg = jnp.dot(x, wg1_ref[...], preferred_element_type=jnp.float32)
    hg1_ref[...] = hg
    als, ald, misc = _gat_prep(hg, as1_ref[...], ad1_ref[...], we1_ref[...],
                               ae1_ref[...], ew_ref[...])
    als_ref[...] = als
    ald_ref[...] = ald
    misc_ref[...] = misc


def _t2_body(adj_ref, dinv_ref, xa1_ref, hg1_ref, e1_ref,
             w2a_ref, b2a_ref, wg2_ref, as2_ref, ad2_ref, we2_ref, ae2_ref,
             bg1_ref, ew_ref,
             xa2_ref, hg2_ref, als_ref, ald_ref, misc_ref):
    adj = adj_ref[...]
    dinv = dinv_ref[...]
    e1 = e1_ref[...]
    denom = jnp.sum(e1, axis=1, keepdims=True) + EPS
    agg = jnp.dot(e1, hg1_ref[...], preferred_element_type=jnp.float32)
    xd1 = jnp.maximum(agg / denom + bg1_ref[...], 0.0)
    h2 = jnp.dot(xa1_ref[...], w2a_ref[...], preferred_element_type=jnp.float32)
    xa2_ref[...] = jnp.maximum(_gcn_dense(adj, dinv, h2) + b2a_ref[...], 0.0)
    hg = jnp.dot(xd1, wg2_ref[...], preferred_element_type=jnp.float32)
    hg2_ref[...] = hg
    als, ald, misc = _gat_prep(hg, as2_ref[...], ad2_ref[...], we2_ref[...],
                               ae2_ref[...], ew_ref[...])
    als_ref[...] = als
    ald_ref[...] = ald
    misc_ref[...] = misc


def _t3_body(adj_ref, dinv_ref, xa2_ref, hg2_ref, e2_ref,
             bg2_ref, w3_ref, b3_ref, p1a_ref, p1b_ref, pb1_ref,
             a_ref, bt_ref):
    adj = adj_ref[...]
    dinv = dinv_ref[...]
    e2 = e2_ref[...]
    denom = jnp.sum(e2, axis=1, keepdims=True) + EPS
    agg = jnp.dot(e2, hg2_ref[...], preferred_element_type=jnp.float32)
    xd2 = jnp.maximum(agg / denom + bg2_ref[...], 0.0)
    xc = ALPHA * xa2_ref[...] + (1.0 - ALPHA) * xd2
    h3 = jnp.dot(xc, w3_ref[...], preferred_element_type=jnp.float32)
    hh = jnp.maximum(_gcn_dense(adj, dinv, h3) + b3_ref[...], 0.0)
    a_ref[...] = (
        jnp.dot(hh, p1a_ref[...], preferred_element_type=jnp.float32)
        + pb1_ref[...]
    )
    bt_ref[...] = jax.lax.dot_general(
        p1b_ref[...], hh, (((0,), (1,)), ((), ())),
        preferred_element_type=jnp.float32)


def _pair_body(a_ref, bt_ref, p2_ref, pb2_ref, out_ref):
    a = a_ref[...]
    bt = bt_ref[...]
    acc = jnp.zeros(out_ref.shape, jnp.float32)
    for k in range(H):
        acc = acc + jnp.maximum(a[:, k:k + 1] + bt[k:k + 1, :], 0.0) \
            * p2_ref[k:k + 1, :]
    z = acc + pb2_ref[...]
    out_ref[...] = 1.0 / (1.0 + jnp.exp(-z))


_NC = 2
_NS = 16
_NW = _NC * _NS
_EF = E + N
_EPT = _EF // _NS
_RPW = N // _NW
_CPB = 16


def _sc_mesh():
    return plsc.VectorSubcoreMesh(core_axis_name="c", subcore_axis_name="s")


def _scatter_phase(pv, valv, buf, wid, ones=None):
    row0 = wid * _RPW

    @plsc.parallel_loop(0, _EF // 16, unroll=8)
    def body(i):
        off = i * 16
        p16 = pv[pl.ds(off, 16)]
        s16 = jnp.bitwise_and(p16, 0xFFFF)
        d16 = jnp.right_shift(p16, 16)
        rows = d16 - row0
        mask = (rows >= 0) & (rows < _RPW)
        v16 = ones if ones is not None else valv[pl.ds(off, 16)]
        plsc.addupdate_scatter(buf, [rows, s16], v16, mask=mask)


def _adj_pallas(sd_packed, zeros2d):
    @functools.partial(
        pl.kernel,
        out_type=jax.ShapeDtypeStruct((N, N), jnp.float32),
        mesh=_sc_mesh(),
        compiler_params=pltpu.CompilerParams(needs_layout_passes=False),
        scratch_types=[
            pltpu.VMEM((_EF,), jnp.int32),
            pltpu.VMEM((_RPW, N), jnp.float32),
            pltpu.SemaphoreType.DMA,
        ],
    )
    def k(p_ref, z_ref, out_ref, pv, buf, sem):
        cid = lax.axis_index("c")
        sid = lax.axis_index("s")
        wid = sid * _NC + cid
        loads = [
            pltpu.async_copy(p_ref, pv, sem),
            pltpu.async_copy(z_ref.at[pl.ds(wid * _RPW, _RPW), :], buf, sem),
        ]
        for c in loads:
            c.wait()
        ones = jnp.full((16,), 1.0, jnp.float32)
        _scatter_phase(pv, None, buf, wid, ones=ones)
        pltpu.sync_copy(buf, out_ref.at[pl.ds(wid * _RPW, _RPW), :])

    return k(sd_packed, zeros2d)


def _gat_pallas(sd_packed, ew_pad, als, ald, misc, zeros2d):
    @functools.partial(
        pl.kernel,
        out_type=jax.ShapeDtypeStruct((N, N), jnp.float32),
        mesh=_sc_mesh(),
        compiler_params=pltpu.CompilerParams(needs_layout_passes=False),
        scratch_types=[
            pltpu.VMEM((_EF,), jnp.int32),
            pltpu.VMEM((_EF,), jnp.float32),
            pltpu.VMEM((_EPT,), jnp.float32),
            pltpu.VMEM((_EPT,), jnp.float32),
            pltpu.VMEM((_RPW, N), jnp.float32),
            pltpu.VMEM((N,), jnp.float32),
            pltpu.VMEM((N,), jnp.float32),
            pltpu.VMEM((128,), jnp.float32),
            pltpu.VMEM_SHARED((_EF,), jnp.float32),
            pltpu.SemaphoreType.DMA,
        ],
    )
    def k(p_ref, ew_ref, als_ref, ald_ref, misc_ref, z_ref, out_ref,
          pv, valv, myv, ewv, buf, alsv, aldv, miscv, xch, sem):
        cid = lax.axis_index("c")
        sid = lax.axis_index("s")
        wid = sid * _NC + cid
        ebase = sid * _EPT
        loads = [
            pltpu.async_copy(p_ref, pv, sem),
            pltpu.async_copy(ew_ref.at[pl.ds(ebase, _EPT)], ewv, sem),
            pltpu.async_copy(als_ref, alsv, sem),
            pltpu.async_copy(ald_ref, aldv, sem),
            pltpu.async_copy(misc_ref, miscv, sem),
            pltpu.async_copy(z_ref.at[pl.ds(wid * _RPW, _RPW), :], buf, sem),
        ]
        for c in loads:
            c.wait()
        c16 = miscv[pl.ds(0, 16)]
        selfe16 = miscv[pl.ds(16, 16)]
        cb16 = miscv[pl.ds(32, 16)]
        lanes = lax.iota(jnp.int32, 16)

        @plsc.parallel_loop(0, _EPT // 16, unroll=8)
        def val_body(i):
            off = i * 16
            p16 = pv[pl.ds(ebase + off, 16)]
            s16 = jnp.bitwise_and(p16, 0xFFFF)
            d16 = jnp.right_shift(p16, 16)
            ew16 = ewv[pl.ds(off, 16)]
            gi = ebase + off + lanes
            as16 = plsc.load_gather(alsv, [s16])
            ad16 = plsc.load_gather(aldv, [d16])
            ale = jnp.where(gi < E, c16 * ew16, selfe16)
            pre = as16 + ad16 + ale
            alpha = jnp.maximum(pre, NEG_SLOPE * pre)
            myv[pl.ds(off, 16)] = jnp.exp(alpha - cb16)
        pltpu.sync_copy(myv, xch.at[pl.ds(ebase, _EPT)])
        plsc.subcore_barrier()
        pltpu.sync_copy(xch, valv)

        _scatter_phase(pv, valv, buf, wid)
        pltpu.sync_copy(buf, out_ref.at[pl.ds(wid * _RPW, _RPW), :])

    return k(sd_packed, ew_pad, als, ald, misc, zeros2d)


def kernel(x, edge_index_above, edge_index_distance, edge_weights_distance,
           W1a, b1a, Wg1, as1, ad1, We1, ae1, bg1,
           W2a, b2a, Wg2, as2, ad2, We2, ae2, bg2,
           W3, b3, P1, pb1, P2, pb2):
    f32 = jnp.float32
    sa, da = edge_index_above[0], edge_index_above[1]
    sd, dd = edge_index_distance[0], edge_index_distance[1]
    loop = jnp.arange(N, dtype=sa.dtype)
    sa_f = jnp.concatenate([sa, loop])
    da_f = jnp.concatenate([da, loop])
    sd_f = jnp.concatenate([sd, loop])
    dd_f = jnp.concatenate([dd, loop])
    ew = edge_weights_distance
    ew2d = ew.reshape(E // 128, 128)
    ew_pad = jnp.concatenate([ew, jnp.zeros((N,), f32)])
    zeros2d = jnp.zeros((N, N), f32)
    sd_pack_a = jnp.left_shift(da_f, 16) | sa_f
    sd_pack_d = jnp.left_shift(dd_f, 16) | sd_f

    adj = _adj_pallas(sd_pack_a, zeros2d)

    b1a_2 = b1a.reshape(1, H)
    bg1_2 = bg1.reshape(1, H)
    b2a_2 = b2a.reshape(1, H)
    bg2_2 = bg2.reshape(1, H)
    b3_2 = b3.reshape(1, H)
    pb1_2 = pb1.reshape(1, H)
    pb2_2 = pb2.reshape(1, 1)
    as1_2, ad1_2 = as1.reshape(H, 1), ad1.reshape(H, 1)
    as2_2, ad2_2 = as2.reshape(H, 1), ad2.reshape(H, 1)
    ae1_2, ae2_2 = ae1.reshape(H, 1), ae2.reshape(H, 1)

    t1 = pl.pallas_call(
        _t1_body,
        out_shape=(
            jax.ShapeDtypeStruct((N, H), f32),
            jax.ShapeDtypeStruct((N, H), f32),
            jax.ShapeDtypeStruct((N, 1), f32),
            jax.ShapeDtypeStruct((N, 1), f32),
            jax.ShapeDtypeStruct((N, 1), f32),
            jax.ShapeDtypeStruct((8, 16), f32),
        ),
    )
    xa1, hg1, als1, ald1, dinv, misc1 = t1(
        x, adj, W1a, b1a_2, Wg1, as1_2, ad1_2, We1, ae1_2, ew2d)

    e1 = _gat_pallas(sd_pack_d, ew_pad, als1.reshape(N), ald1.reshape(N),
                     misc1.reshape(128), zeros2d)

    t2 = pl.pallas_call(
        _t2_body,
        out_shape=(
            jax.ShapeDtypeStruct((N, H), f32),
            jax.ShapeDtypeStruct((N, H), f32),
            jax.ShapeDtypeStruct((N, 1), f32),
            jax.ShapeDtypeStruct((N, 1), f32),
            jax.ShapeDtypeStruct((8, 16), f32),
        ),
    )
    xa2, hg2, als2, ald2, misc2 = t2(
        adj, dinv, xa1, hg1, e1, W2a, b2a_2, Wg2, as2_2, ad2_2, We2, ae2_2,
        bg1_2, ew2d)

    e2 = _gat_pallas(sd_pack_d, ew_pad, als2.reshape(N), ald2.reshape(N),
                     misc2.reshape(128), zeros2d)

    t3 = pl.pallas_call(
        _t3_body,
        out_shape=(
            jax.ShapeDtypeStruct((N, H), f32),
            jax.ShapeDtypeStruct((H, N), f32),
        ),
    )
    a_mat, bt_mat = t3(
        adj, dinv, xa2, hg2, e2, bg2_2, W3, b3_2, P1[:H], P1[H:], pb1_2)

    BI = 256
    pair = pl.pallas_call(
        _pair_body,
        grid=(N // BI,),
        in_specs=[
            pl.BlockSpec((BI, H), lambda i: (i, 0)),
            pl.BlockSpec((H, N), lambda i: (0, 0)),
            pl.BlockSpec((H, 1), lambda i: (0, 0)),
            pl.BlockSpec((1, 1), lambda i: (0, 0)),
        ],
        out_specs=pl.BlockSpec((BI, N), lambda i: (i, 0)),
        out_shape=jax.ShapeDtypeStruct((N, N), f32),
    )
    out2d = pair(a_mat, bt_mat, P2, pb2_2)
    return out2d.reshape(N * N)
